# baseline (speedup 1.0000x reference)
import functools

import jax
import jax.numpy as jnp
from jax import lax
from jax.experimental import pallas as pl
from jax.experimental.pallas import tpu as pltpu
from jax.experimental.pallas import tpu_sc as plsc

_VOCAB = 100000
_EMB = 128
_HID = 768
_N_BLOCKS = 16
_BLOCK_SIZE = 128
_B = 32
_T = 2048
_EPS = 1e-12

_TOKENS = _B * _T
_CHUNK = 128
_NW = 32
_PER_W = _TOKENS // _NW
_N_CHUNKS = _PER_W // _CHUNK


def _sc_gather(x3, table, n_tokens):
    per_w = n_tokens // _NW
    n_chunks = per_w // _CHUNK
    mesh = plsc.VectorSubcoreMesh(core_axis_name="c", subcore_axis_name="s")

    @functools.partial(
        pl.kernel,
        mesh=mesh,
        out_type=jax.ShapeDtypeStruct((n_tokens, _EMB), jnp.float32),
        scratch_types=[
            pltpu.VMEM((n_chunks, _CHUNK), jnp.int32),
            pltpu.VMEM((2, _CHUNK, _EMB), jnp.float32),
            pltpu.SemaphoreType.DMA,
            pltpu.SemaphoreType.DMA,
            pltpu.SemaphoreType.DMA,
        ],
    )
    def k(x_hbm, table_hbm, out_hbm, idx_v, rows_v, gsem0, gsem1, wsem):
        wid = lax.axis_index("s") * 2 + lax.axis_index("c")
        pltpu.sync_copy(x_hbm.at[wid], idx_v)

        gsems = [gsem0, gsem1]

        def start_gather(c, buf):
            return pltpu.async_copy(
                table_hbm.at[idx_v.at[c]], rows_v.at[buf], gsems[buf]
            )

        handles = [None, None]
        handles[0] = start_gather(0, 0)
        for c in range(n_chunks):
            buf = c % 2
            if c + 1 < n_chunks:
                handles[1 - buf] = start_gather(c + 1, 1 - buf)
            handles[buf].wait()
            out = pltpu.async_copy(
                rows_v.at[buf],
                out_hbm.at[pl.ds((wid * per_w) + c * _CHUNK, _CHUNK), :],
                wsem,
            )
            out.wait()

    return k(x3, table)


_PARTS = [(16384, 4096), (49152, 4096)]


def _proj_ln_part(emb, W16, pos_table, block3, inner_table, b2, gamma, beta,
                  tok_start, tb, prev):
    n_steps = emb.shape[0] // tb
    seq_per_step = tb // _T
    off = tok_start // tb

    def body(emb_ref, w_ref, pos_ref, blk_ref, inner_ref, b_ref,
             gamma_ref, beta_ref, *rest):
        out_ref = rest[-1]
        a = emb_ref[...].astype(jnp.bfloat16)
        h = jnp.dot(a, w_ref[...], preferred_element_type=jnp.float32)
        bias = (pos_ref[...].reshape(_N_BLOCKS, _BLOCK_SIZE, _HID)
                + blk_ref[...] + inner_ref[...][None] + b_ref[...][None])
        h = (h.reshape(seq_per_step, _T, _HID)
             + bias.reshape(_T, _HID)[None]).reshape(tb, _HID)
        u = jnp.mean(h, axis=-1, keepdims=True)
        d = h - u
        s = jnp.mean(d * d, axis=-1, keepdims=True)
        out_ref[...] = gamma_ref[...] * (d * lax.rsqrt(s + _EPS)) + beta_ref[...]

    in_specs = [
        pl.BlockSpec((tb, _EMB), lambda i: (i, 0)),
        pl.BlockSpec((_EMB, _HID), lambda i: (0, 0)),
        pl.BlockSpec((_T, _HID), lambda i: (0, 0)),
        pl.BlockSpec((_N_BLOCKS, 1, _HID), lambda i: (0, 0, 0)),
        pl.BlockSpec((_BLOCK_SIZE, _HID), lambda i: (0, 0)),
        pl.BlockSpec((1, _HID), lambda i: (0, 0)),
        pl.BlockSpec((1, _HID), lambda i: (0, 0)),
        pl.BlockSpec((1, _HID), lambda i: (0, 0)),
    ]
    args = [emb, W16, pos_table, block3, inner_table, b2,
            gamma.reshape(1, _HID), beta.reshape(1, _HID)]
    aliases = {}
    if prev is not None:
        in_specs.append(pl.BlockSpec(memory_space=pl.ANY))
        args.append(prev)
        aliases = {8: 0}

    return pl.pallas_call(
        body,
        grid=(n_steps,),
        in_specs=in_specs,
        out_specs=pl.BlockSpec((tb, _HID), lambda i: (off + i, 0)),
        out_shape=jax.ShapeDtypeStruct((_TOKENS, _HID), jnp.float32),
        input_output_aliases=aliases,
    )(*args)


def kernel(x, table, W, b, gamma, beta, pos_table, block_table, inner_table):
    x_flat = x.reshape(_TOKENS)
    W16 = W.astype(jnp.bfloat16)
    block3 = block_table.reshape(_N_BLOCKS, 1, _HID)
    b2 = b.reshape(1, _HID)
    embs = []
    start = 0
    for size, _ in _PARTS:
        x3 = x_flat[start:start + size].reshape(_NW, size // (_NW * _CHUNK),
                                                _CHUNK)
        embs.append(_sc_gather(x3, table, size))
        start += size
    out = None
    start = 0
    for (size, tb), emb in zip(_PARTS, embs):
        out = _proj_ln_part(emb, W16, pos_table, block3, inner_table, b2,
                            gamma, beta, start, tb, out)
        start += size
    return out.reshape(_B, _T, _HID)


# speedup vs baseline: 1.0044x; 1.0044x over previous
import functools

import jax
import jax.numpy as jnp
from jax import lax
from jax.experimental import pallas as pl
from jax.experimental.pallas import tpu as pltpu
from jax.experimental.pallas import tpu_sc as plsc

_VOCAB = 100000
_EMB = 128
_HID = 768
_N_BLOCKS = 16
_BLOCK_SIZE = 128
_B = 32
_T = 2048
_EPS = 1e-12

_TOKENS = _B * _T
_CHUNK = 128
_NW = 32
_PER_W = _TOKENS // _NW
_N_CHUNKS = _PER_W // _CHUNK


def _sc_gather(x3, table, n_tokens):
    per_w = n_tokens // _NW
    n_chunks = per_w // _CHUNK
    mesh = plsc.VectorSubcoreMesh(core_axis_name="c", subcore_axis_name="s")

    @functools.partial(
        pl.kernel,
        mesh=mesh,
        out_type=jax.ShapeDtypeStruct((n_tokens, _EMB), jnp.float32),
        scratch_types=[
            pltpu.VMEM((n_chunks, _CHUNK), jnp.int32),
            pltpu.VMEM((2, _CHUNK, _EMB), jnp.float32),
            pltpu.SemaphoreType.DMA,
            pltpu.SemaphoreType.DMA,
            pltpu.SemaphoreType.DMA,
        ],
    )
    def k(x_hbm, table_hbm, out_hbm, idx_v, rows_v, gsem0, gsem1, wsem):
        wid = lax.axis_index("s") * 2 + lax.axis_index("c")
        pltpu.sync_copy(x_hbm.at[wid], idx_v)

        gsems = [gsem0, gsem1]

        def start_gather(c, buf):
            return pltpu.async_copy(
                table_hbm.at[idx_v.at[c]], rows_v.at[buf], gsems[buf]
            )

        handles = [None, None]
        handles[0] = start_gather(0, 0)
        for c in range(n_chunks):
            buf = c % 2
            if c + 1 < n_chunks:
                handles[1 - buf] = start_gather(c + 1, 1 - buf)
            handles[buf].wait()
            out = pltpu.async_copy(
                rows_v.at[buf],
                out_hbm.at[pl.ds((wid * per_w) + c * _CHUNK, _CHUNK), :],
                wsem,
            )
            out.wait()

    return k(x3, table)


_PARTS = [(12288, 2048), (53248, 4096)]


def _proj_ln_part(emb, W16, pos_table, block3, inner_table, b2, gamma, beta,
                  tok_start, tb, prev):
    n_steps = emb.shape[0] // tb
    seq_per_step = tb // _T
    off = tok_start // tb

    def body(emb_ref, w_ref, pos_ref, blk_ref, inner_ref, b_ref,
             gamma_ref, beta_ref, *rest):
        out_ref = rest[-1]
        a = emb_ref[...].astype(jnp.bfloat16)
        h = jnp.dot(a, w_ref[...], preferred_element_type=jnp.float32)
        bias = (pos_ref[...].reshape(_N_BLOCKS, _BLOCK_SIZE, _HID)
                + blk_ref[...] + inner_ref[...][None] + b_ref[...][None])
        h = (h.reshape(seq_per_step, _T, _HID)
             + bias.reshape(_T, _HID)[None]).reshape(tb, _HID)
        u = jnp.mean(h, axis=-1, keepdims=True)
        d = h - u
        s = jnp.mean(d * d, axis=-1, keepdims=True)
        out_ref[...] = gamma_ref[...] * (d * lax.rsqrt(s + _EPS)) + beta_ref[...]

    in_specs = [
        pl.BlockSpec((tb, _EMB), lambda i: (i, 0)),
        pl.BlockSpec((_EMB, _HID), lambda i: (0, 0)),
        pl.BlockSpec((_T, _HID), lambda i: (0, 0)),
        pl.BlockSpec((_N_BLOCKS, 1, _HID), lambda i: (0, 0, 0)),
        pl.BlockSpec((_BLOCK_SIZE, _HID), lambda i: (0, 0)),
        pl.BlockSpec((1, _HID), lambda i: (0, 0)),
        pl.BlockSpec((1, _HID), lambda i: (0, 0)),
        pl.BlockSpec((1, _HID), lambda i: (0, 0)),
    ]
    args = [emb, W16, pos_table, block3, inner_table, b2,
            gamma.reshape(1, _HID), beta.reshape(1, _HID)]
    aliases = {}
    if prev is not None:
        in_specs.append(pl.BlockSpec(memory_space=pl.ANY))
        args.append(prev)
        aliases = {8: 0}

    return pl.pallas_call(
        body,
        grid=(n_steps,),
        in_specs=in_specs,
        out_specs=pl.BlockSpec((tb, _HID), lambda i: (off + i, 0)),
        out_shape=jax.ShapeDtypeStruct((_TOKENS, _HID), jnp.float32),
        input_output_aliases=aliases,
    )(*args)


def kernel(x, table, W, b, gamma, beta, pos_table, block_table, inner_table):
    x_flat = x.reshape(_TOKENS)
    W16 = W.astype(jnp.bfloat16)
    block3 = block_table.reshape(_N_BLOCKS, 1, _HID)
    b2 = b.reshape(1, _HID)
    embs = []
    start = 0
    for size, _ in _PARTS:
        x3 = x_flat[start:start + size].reshape(_NW, size // (_NW * _CHUNK),
                                                _CHUNK)
        embs.append(_sc_gather(x3, table, size))
        start += size
    out = None
    start = 0
    for (size, tb), emb in zip(_PARTS, embs):
        out = _proj_ln_part(emb, W16, pos_table, block3, inner_table, b2,
                            gamma, beta, start, tb, out)
        start += size
    return out.reshape(_B, _T, _HID)


# speedup vs baseline: 1.0113x; 1.0069x over previous
import functools

import jax
import jax.numpy as jnp
from jax import lax
from jax.experimental import pallas as pl
from jax.experimental.pallas import tpu as pltpu
from jax.experimental.pallas import tpu_sc as plsc

_VOCAB = 100000
_EMB = 128
_HID = 768
_N_BLOCKS = 16
_BLOCK_SIZE = 128
_B = 32
_T = 2048
_EPS = 1e-12

_TOKENS = _B * _T
_CHUNK = 128
_NW = 32
_PER_W = _TOKENS // _NW
_N_CHUNKS = _PER_W // _CHUNK


def _sc_gather(x3, table, n_tokens):
    per_w = n_tokens // _NW
    n_chunks = per_w // _CHUNK
    mesh = plsc.VectorSubcoreMesh(core_axis_name="c", subcore_axis_name="s")

    @functools.partial(
        pl.kernel,
        mesh=mesh,
        out_type=jax.ShapeDtypeStruct((n_tokens, _EMB), jnp.float32),
        scratch_types=[
            pltpu.VMEM((n_chunks, _CHUNK), jnp.int32),
            pltpu.VMEM((2, _CHUNK, _EMB), jnp.float32),
            pltpu.SemaphoreType.DMA,
            pltpu.SemaphoreType.DMA,
            pltpu.SemaphoreType.DMA,
        ],
    )
    def k(x_hbm, table_hbm, out_hbm, idx_v, rows_v, gsem0, gsem1, wsem):
        wid = lax.axis_index("s") * 2 + lax.axis_index("c")
        pltpu.sync_copy(x_hbm.at[wid], idx_v)

        gsems = [gsem0, gsem1]

        def start_gather(c, buf):
            return pltpu.async_copy(
                table_hbm.at[idx_v.at[c]], rows_v.at[buf], gsems[buf]
            )

        handles = [None, None]
        handles[0] = start_gather(0, 0)
        for c in range(n_chunks):
            buf = c % 2
            if c + 1 < n_chunks:
                handles[1 - buf] = start_gather(c + 1, 1 - buf)
            handles[buf].wait()
            out = pltpu.async_copy(
                rows_v.at[buf],
                out_hbm.at[pl.ds((wid * per_w) + c * _CHUNK, _CHUNK), :],
                wsem,
            )
            out.wait()

    return k(x3, table)


_PARTS = [(12288, 4096), (53248, 4096)]


def _proj_ln_part(emb, W16, pos_table, block3, inner_table, b2, gamma, beta,
                  tok_start, tb, prev):
    n_steps = emb.shape[0] // tb
    seq_per_step = tb // _T
    off = tok_start // tb

    def body(emb_ref, w_ref, pos_ref, blk_ref, inner_ref, b_ref,
             gamma_ref, beta_ref, *rest):
        out_ref = rest[-1]
        a = emb_ref[...].astype(jnp.bfloat16)
        h = jnp.dot(a, w_ref[...], preferred_element_type=jnp.float32)
        bias = (pos_ref[...].reshape(_N_BLOCKS, _BLOCK_SIZE, _HID)
                + blk_ref[...] + inner_ref[...][None] + b_ref[...][None])
        h = (h.reshape(seq_per_step, _T, _HID)
             + bias.reshape(_T, _HID)[None]).reshape(tb, _HID)
        u = jnp.mean(h, axis=-1, keepdims=True)
        d = h - u
        s = jnp.mean(d * d, axis=-1, keepdims=True)
        out_ref[...] = gamma_ref[...] * (d * lax.rsqrt(s + _EPS)) + beta_ref[...]

    in_specs = [
        pl.BlockSpec((tb, _EMB), lambda i: (i, 0)),
        pl.BlockSpec((_EMB, _HID), lambda i: (0, 0)),
        pl.BlockSpec((_T, _HID), lambda i: (0, 0)),
        pl.BlockSpec((_N_BLOCKS, 1, _HID), lambda i: (0, 0, 0)),
        pl.BlockSpec((_BLOCK_SIZE, _HID), lambda i: (0, 0)),
        pl.BlockSpec((1, _HID), lambda i: (0, 0)),
        pl.BlockSpec((1, _HID), lambda i: (0, 0)),
        pl.BlockSpec((1, _HID), lambda i: (0, 0)),
    ]
    args = [emb, W16, pos_table, block3, inner_table, b2,
            gamma.reshape(1, _HID), beta.reshape(1, _HID)]
    aliases = {}
    if prev is not None:
        in_specs.append(pl.BlockSpec(memory_space=pl.ANY))
        args.append(prev)
        aliases = {8: 0}

    return pl.pallas_call(
        body,
        grid=(n_steps,),
        in_specs=in_specs,
        out_specs=pl.BlockSpec((tb, _HID), lambda i: (off + i, 0)),
        out_shape=jax.ShapeDtypeStruct((_TOKENS, _HID), jnp.float32),
        input_output_aliases=aliases,
    )(*args)


def kernel(x, table, W, b, gamma, beta, pos_table, block_table, inner_table):
    x_flat = x.reshape(_TOKENS)
    W16 = W.astype(jnp.bfloat16)
    block3 = block_table.reshape(_N_BLOCKS, 1, _HID)
    b2 = b.reshape(1, _HID)
    embs = []
    start = 0
    for size, _ in _PARTS:
        x3 = x_flat[start:start + size].reshape(_NW, size // (_NW * _CHUNK),
                                                _CHUNK)
        embs.append(_sc_gather(x3, table, size))
        start += size
    out = None
    start = 0
    for (size, tb), emb in zip(_PARTS, embs):
        out = _proj_ln_part(emb, W16, pos_table, block3, inner_table, b2,
                            gamma, beta, start, tb, out)
        start += size
    return out.reshape(_B, _T, _HID)
